# baseline (speedup 1.0000x reference)
import functools

import jax
import jax.numpy as jnp
from jax import lax
from jax.experimental import pallas as pl
from jax.experimental.pallas import tpu as pltpu
from jax.experimental.pallas import tpu_sc as plsc

_DIM = 256
_STATS_TILE = 8000
_TTILE = 1024
_NCOORD = 41
_TBL = 69632
_CB = 16


def _stats_idx_body(coorst_ref, offc_ref, winc_ref, out_ref, idx_ref):
    i = pl.program_id(0)
    ci = coorst_ref[...]
    idx_ref[...] = (ci[:, 0:1, :] * (_NCOORD * _NCOORD)
                    + ci[:, 1:2, :] * _NCOORD + ci[:, 2:3, :])
    c = ci.astype(jnp.float32)
    t = c - offc_ref[...]
    t = t / winc_ref[...]
    t = t * 2.0
    t = t * 3.1415
    lb = t.astype(jnp.bfloat16).astype(jnp.float32)

    @pl.when(i == 0)
    def _init():
        out_ref[...] = jnp.zeros_like(out_ref)

    for j in range(3):
        out_ref[3:4, j:j + 1] += jnp.sum(
            lb[:, j, :], axis=1, keepdims=True)
        for k in range(j, 3):
            p = jnp.sum(lb[:, j, :] * lb[:, k, :], axis=1, keepdims=True)
            out_ref[j:j + 1, k:k + 1] += p


def _loc_bf16(c, off, win):
    t = c.astype(jnp.float32) - off
    t = t / win
    t = t * 2.0
    t = t * 3.1415
    return t.astype(jnp.bfloat16).astype(jnp.float32)


def _table_body(w1f_ref, b1_ref, mean_ref, scale_ref, beta_ref, w2_ref,
                b2_ref, out_ref):
    i = pl.program_id(0)
    r = lax.broadcasted_iota(jnp.int32, (_TTILE, 1), 0) + i * _TTILE
    c0 = r // (_NCOORD * _NCOORD)
    rem = r - c0 * (_NCOORD * _NCOORD)
    c1 = rem // _NCOORD
    c2 = rem - c1 * _NCOORD
    l0 = _loc_bf16(c0, 20.5, 41.0)
    l1 = _loc_bf16(c1, 720.0, 1440.0)
    l2 = _loc_bf16(c2, 720.0, 1440.0)
    h = (l0 * w1f_ref[0:1, :] + l1 * w1f_ref[1:2, :]
         + l2 * w1f_ref[2:3, :]) + b1_ref[...]
    hn = (h - mean_ref[...]) * scale_ref[...] + beta_ref[...]
    u = jnp.maximum(hn, 0.0).astype(jnp.bfloat16)
    out_ref[...] = jnp.dot(u, w2_ref[...],
                           preferred_element_type=jnp.float32) + b2_ref[...]


def _make_sc_gather_add(n, dim):
    info = plsc.get_sparse_core_info()
    nw = info.num_cores * info.num_subcores
    rows_main = 6256
    n_last = n - (nw - 1) * rows_main
    mesh = plsc.VectorSubcoreMesh(core_axis_name="c", subcore_axis_name="s")

    @functools.partial(
        pl.kernel, mesh=mesh,
        out_type=jax.ShapeDtypeStruct((n, dim), jnp.float32),
        scratch_types=[
            pltpu.VMEM((_CB,), jnp.int32),
            pltpu.VMEM((_CB, dim), jnp.float32),
            pltpu.VMEM((_CB, dim), jnp.float32),
            pltpu.SemaphoreType.DMA,
        ],
    )
    def sc_kernel(table_hbm, idx_hbm, feat_hbm, out_hbm,
                  idx_v, rows_v, feat_v, sem):
        wid = lax.axis_index("s") * info.num_cores + lax.axis_index("c")
        start = wid * rows_main
        nch = jnp.where(wid < nw - 1, rows_main // _CB, n_last // _CB)

        def body(j, carry):
            base = start + j * _CB
            pltpu.sync_copy(idx_hbm.at[pl.ds(base, _CB)], idx_v)
            pltpu.sync_copy(feat_hbm.at[pl.ds(base, _CB)], feat_v)
            pltpu.async_copy(table_hbm.at[idx_v], rows_v, sem).wait()
            for rr in range(_CB):
                for cc in range(dim // 16):
                    sl = (rr, pl.ds(cc * 16, 16))
                    rows_v[sl] = rows_v[sl] + feat_v[sl]
            pltpu.sync_copy(rows_v, out_hbm.at[pl.ds(base, _CB)])
            return carry

        lax.fori_loop(0, nch, body, 0)

    return sc_kernel


def kernel(features, coors, W1, b1, gamma, beta, W2, b2):
    n = features.shape[0]
    nf = jnp.float32(n)

    w1r = W1[::-1].astype(jnp.bfloat16)
    off = jnp.array([[20.5, 720.0, 720.0]], dtype=jnp.float32)
    win = jnp.array([[41.0, 1440.0, 1440.0]], dtype=jnp.float32)

    nchunk = n // _STATS_TILE
    coorst = coors.reshape(nchunk, _STATS_TILE, 3).transpose(0, 2, 1)
    offc = off.reshape(1, 3, 1)
    winc = win.reshape(1, 3, 1)
    stats, idx3 = pl.pallas_call(
        _stats_idx_body,
        grid=(nchunk,),
        in_specs=[
            pl.BlockSpec((1, 3, _STATS_TILE), lambda i: (i, 0, 0)),
            pl.BlockSpec((1, 3, 1), lambda i: (0, 0, 0)),
            pl.BlockSpec((1, 3, 1), lambda i: (0, 0, 0)),
        ],
        out_specs=[
            pl.BlockSpec((4, 3), lambda i: (0, 0)),
            pl.BlockSpec((1, 1, _STATS_TILE), lambda i: (i, 0, 0)),
        ],
        out_shape=[
            jax.ShapeDtypeStruct((4, 3), jnp.float32),
            jax.ShapeDtypeStruct((nchunk, 1, _STATS_TILE), jnp.int32),
        ],
        compiler_params=pltpu.CompilerParams(
            dimension_semantics=("arbitrary",)),
    )(coorst, offc, winc)
    idx = idx3.reshape(n)

    m = stats[3, :] / nf
    mom = stats[0:3, :] / nf
    sym = mom + mom.T - jnp.diag(jnp.diag(mom))
    cov = sym - jnp.outer(m, m)
    w1f = w1r.astype(jnp.float32)
    mean = (jnp.dot(m, w1f, precision=jax.lax.Precision.HIGHEST)
            + b1)[None, :]
    var = jnp.einsum("jc,jk,kc->c", w1f, cov, w1f,
                     precision=jax.lax.Precision.HIGHEST)[None, :]
    scale = gamma[None, :] / jnp.sqrt(var + 1e-5)

    table = pl.pallas_call(
        _table_body,
        grid=(_TBL // _TTILE,),
        in_specs=[
            pl.BlockSpec((3, _DIM), lambda i: (0, 0)),
            pl.BlockSpec((1, _DIM), lambda i: (0, 0)),
            pl.BlockSpec((1, _DIM), lambda i: (0, 0)),
            pl.BlockSpec((1, _DIM), lambda i: (0, 0)),
            pl.BlockSpec((1, _DIM), lambda i: (0, 0)),
            pl.BlockSpec((_DIM, _DIM), lambda i: (0, 0)),
            pl.BlockSpec((1, _DIM), lambda i: (0, 0)),
        ],
        out_specs=pl.BlockSpec((_TTILE, _DIM), lambda i: (i, 0)),
        out_shape=jax.ShapeDtypeStruct((_TBL, _DIM), jnp.float32),
        compiler_params=pltpu.CompilerParams(
            dimension_semantics=("parallel",)),
    )(w1f, b1[None, :], mean, scale, beta[None, :],
      W2.astype(jnp.bfloat16), b2[None, :])

    return _make_sc_gather_add(n, _DIM)(table, idx, features)


# speedup vs baseline: 3.4651x; 3.4651x over previous
import functools

import jax
import jax.numpy as jnp
from jax import lax
from jax.experimental import pallas as pl
from jax.experimental.pallas import tpu as pltpu
from jax.experimental.pallas import tpu_sc as plsc

_DIM = 256
_STATS_TILE = 8000
_TTILE = 1024
_NCOORD = 41
_TBL = 69632
_CB = 16


def _stats_idx_body(coorst_ref, offc_ref, winc_ref, out_ref, idx_ref):
    i = pl.program_id(0)
    ci = coorst_ref[...]
    idx_ref[...] = (ci[:, 0:1, :] * (_NCOORD * _NCOORD)
                    + ci[:, 1:2, :] * _NCOORD + ci[:, 2:3, :])
    c = ci.astype(jnp.float32)
    t = c - offc_ref[...]
    t = t / winc_ref[...]
    t = t * 2.0
    t = t * 3.1415
    lb = t.astype(jnp.bfloat16).astype(jnp.float32)

    @pl.when(i == 0)
    def _init():
        out_ref[...] = jnp.zeros_like(out_ref)

    for j in range(3):
        out_ref[3:4, j:j + 1] += jnp.sum(
            lb[:, j, :], axis=1, keepdims=True)
        for k in range(j, 3):
            p = jnp.sum(lb[:, j, :] * lb[:, k, :], axis=1, keepdims=True)
            out_ref[j:j + 1, k:k + 1] += p


def _loc_bf16(c, off, win):
    t = c.astype(jnp.float32) - off
    t = t / win
    t = t * 2.0
    t = t * 3.1415
    return t.astype(jnp.bfloat16).astype(jnp.float32)


def _table_body(w1f_ref, b1_ref, mean_ref, scale_ref, beta_ref, w2_ref,
                b2_ref, out_ref):
    i = pl.program_id(0)
    r = lax.broadcasted_iota(jnp.int32, (_TTILE, 1), 0) + i * _TTILE
    c0 = r // (_NCOORD * _NCOORD)
    rem = r - c0 * (_NCOORD * _NCOORD)
    c1 = rem // _NCOORD
    c2 = rem - c1 * _NCOORD
    l0 = _loc_bf16(c0, 20.5, 41.0)
    l1 = _loc_bf16(c1, 720.0, 1440.0)
    l2 = _loc_bf16(c2, 720.0, 1440.0)
    h = (l0 * w1f_ref[0:1, :] + l1 * w1f_ref[1:2, :]
         + l2 * w1f_ref[2:3, :]) + b1_ref[...]
    hn = (h - mean_ref[...]) * scale_ref[...] + beta_ref[...]
    u = jnp.maximum(hn, 0.0).astype(jnp.bfloat16)
    out_ref[...] = jnp.dot(u, w2_ref[...],
                           preferred_element_type=jnp.float32) + b2_ref[...]


_SC_CB = 96
_SC_NCH = 64
_SC_ROWS = _SC_CB * _SC_NCH
_TCB = 16


def _make_sc_gather_add(n, dim):
    info = plsc.get_sparse_core_info()
    nw = info.num_cores * info.num_subcores
    mesh = plsc.VectorSubcoreMesh(core_axis_name="c", subcore_axis_name="s")
    tail_start = nw * _SC_ROWS
    n_tail_ch = (n - tail_start) // _TCB
    hi = n_tail_ch - 6 * nw

    @functools.partial(
        pl.kernel, mesh=mesh,
        out_type=jax.ShapeDtypeStruct((n, dim), jnp.float32),
        scratch_types=[
            pltpu.VMEM((_SC_ROWS,), jnp.int32),
            pltpu.VMEM((_SC_CB, dim), jnp.float32),
            pltpu.VMEM((_SC_CB, dim), jnp.float32),
            pltpu.VMEM((_SC_CB, dim), jnp.float32),
            pltpu.VMEM((_SC_CB, dim), jnp.float32),
            pltpu.VMEM((_TCB,), jnp.int32),
            pltpu.SemaphoreType.DMA,
            pltpu.SemaphoreType.DMA,
            pltpu.SemaphoreType.DMA,
            pltpu.SemaphoreType.DMA,
            pltpu.SemaphoreType.DMA,
            pltpu.SemaphoreType.DMA,
        ],
    )
    def sc_kernel(table_hbm, idx_hbm, feat_hbm, out_hbm,
                  idx_all, rows0, rows1, feat0, feat1, tidx,
                  g0, g1, f0, f1, o0, o1):
        wid = lax.axis_index("s") * info.num_cores + lax.axis_index("c")
        start = wid * _SC_ROWS
        pltpu.sync_copy(idx_hbm.at[pl.ds(start, _SC_ROWS)], idx_all)
        rows = (rows0, rows1)
        feats = (feat0, feat1)
        gs = (g0, g1)
        fs = (f0, f1)
        os_ = (o0, o1)

        def issue(b, j):
            base = start + j * _SC_CB
            pltpu.async_copy(
                table_hbm.at[idx_all.at[pl.ds(j * _SC_CB, _SC_CB)]],
                rows[b], gs[b])
            pltpu.async_copy(feat_hbm.at[pl.ds(base, _SC_CB)],
                             feats[b], fs[b])

        issue(0, 0)
        issue(1, 1)

        def add_rows(rbuf, fbuf, nrows):
            def addrow(rr, c):
                for cc in range(dim // 16):
                    sl = (rr, pl.ds(cc * 16, 16))
                    rbuf[sl] = rbuf[sl] + fbuf[sl]
                return c
            lax.fori_loop(0, nrows, addrow, 0)

        def outer(k, carry):
            for b in range(2):
                j = 2 * k + b
                base = start + j * _SC_CB
                pltpu.make_async_copy(
                    table_hbm.at[idx_all.at[pl.ds(0, _SC_CB)]],
                    rows[b], gs[b]).wait()
                pltpu.make_async_copy(
                    feat_hbm.at[pl.ds(start, _SC_CB)],
                    feats[b], fs[b]).wait()
                add_rows(rows[b], feats[b], _SC_CB)
                pltpu.async_copy(rows[b], out_hbm.at[pl.ds(base, _SC_CB)],
                                 os_[b])
                pltpu.make_async_copy(
                    rows[b], out_hbm.at[pl.ds(base, _SC_CB)],
                    os_[b]).wait()

                @pl.when(j + 2 < _SC_NCH)
                def _():
                    issue(b, j + 2)
            return carry

        lax.fori_loop(0, _SC_NCH // 2, outer, 0)

        cw = jnp.where(wid < hi, 7, 6)
        off_ch = jnp.where(wid < hi, 7 * wid, 7 * hi + 6 * (wid - hi))

        def tbody(t, c):
            base = tail_start + (off_ch + t) * _TCB
            pltpu.sync_copy(idx_hbm.at[pl.ds(base, _TCB)], tidx)
            pltpu.sync_copy(feat_hbm.at[pl.ds(base, _TCB)],
                            feat0.at[pl.ds(0, _TCB)])
            pltpu.async_copy(table_hbm.at[tidx],
                             rows0.at[pl.ds(0, _TCB)], g0).wait()
            add_rows(rows0, feat0, _TCB)
            pltpu.sync_copy(rows0.at[pl.ds(0, _TCB)],
                            out_hbm.at[pl.ds(base, _TCB)])
            return c

        lax.fori_loop(0, cw, tbody, 0)

    return sc_kernel


def kernel(features, coors, W1, b1, gamma, beta, W2, b2):
    n = features.shape[0]
    nf = jnp.float32(n)

    w1r = W1[::-1].astype(jnp.bfloat16)
    off = jnp.array([[20.5, 720.0, 720.0]], dtype=jnp.float32)
    win = jnp.array([[41.0, 1440.0, 1440.0]], dtype=jnp.float32)

    nchunk = n // _STATS_TILE
    coorst = coors.reshape(nchunk, _STATS_TILE, 3).transpose(0, 2, 1)
    offc = off.reshape(1, 3, 1)
    winc = win.reshape(1, 3, 1)
    stats, idx3 = pl.pallas_call(
        _stats_idx_body,
        grid=(nchunk,),
        in_specs=[
            pl.BlockSpec((1, 3, _STATS_TILE), lambda i: (i, 0, 0)),
            pl.BlockSpec((1, 3, 1), lambda i: (0, 0, 0)),
            pl.BlockSpec((1, 3, 1), lambda i: (0, 0, 0)),
        ],
        out_specs=[
            pl.BlockSpec((4, 3), lambda i: (0, 0)),
            pl.BlockSpec((1, 1, _STATS_TILE), lambda i: (i, 0, 0)),
        ],
        out_shape=[
            jax.ShapeDtypeStruct((4, 3), jnp.float32),
            jax.ShapeDtypeStruct((nchunk, 1, _STATS_TILE), jnp.int32),
        ],
        compiler_params=pltpu.CompilerParams(
            dimension_semantics=("arbitrary",)),
    )(coorst, offc, winc)
    idx = idx3.reshape(n)

    m = stats[3, :] / nf
    mom = stats[0:3, :] / nf
    sym = mom + mom.T - jnp.diag(jnp.diag(mom))
    cov = sym - jnp.outer(m, m)
    w1f = w1r.astype(jnp.float32)
    mean = (jnp.dot(m, w1f, precision=jax.lax.Precision.HIGHEST)
            + b1)[None, :]
    var = jnp.einsum("jc,jk,kc->c", w1f, cov, w1f,
                     precision=jax.lax.Precision.HIGHEST)[None, :]
    scale = gamma[None, :] / jnp.sqrt(var + 1e-5)

    table = pl.pallas_call(
        _table_body,
        grid=(_TBL // _TTILE,),
        in_specs=[
            pl.BlockSpec((3, _DIM), lambda i: (0, 0)),
            pl.BlockSpec((1, _DIM), lambda i: (0, 0)),
            pl.BlockSpec((1, _DIM), lambda i: (0, 0)),
            pl.BlockSpec((1, _DIM), lambda i: (0, 0)),
            pl.BlockSpec((1, _DIM), lambda i: (0, 0)),
            pl.BlockSpec((_DIM, _DIM), lambda i: (0, 0)),
            pl.BlockSpec((1, _DIM), lambda i: (0, 0)),
        ],
        out_specs=pl.BlockSpec((_TTILE, _DIM), lambda i: (i, 0)),
        out_shape=jax.ShapeDtypeStruct((_TBL, _DIM), jnp.float32),
        compiler_params=pltpu.CompilerParams(
            dimension_semantics=("parallel",)),
    )(w1f, b1[None, :], mean, scale, beta[None, :],
      W2.astype(jnp.bfloat16), b2[None, :])

    return _make_sc_gather_add(n, _DIM)(table, idx, features)


# speedup vs baseline: 5.5746x; 1.6088x over previous
import jax
import jax.numpy as jnp
from jax.experimental import pallas as pl
from jax.experimental.pallas import tpu as pltpu

_DIM = 256
_STATS_TILE = 8000
_MAIN_TILE = 4000


def _location(coors_ref, off_ref, win_ref):
    c = coors_ref[...].astype(jnp.float32)
    t = c - off_ref[...]
    t = t / win_ref[...]
    t = t * 2.0
    return t * 3.1415


def _h(coors_ref, off_ref, win_ref, w1_ref, b1_ref):
    l = _location(coors_ref, off_ref, win_ref).astype(jnp.bfloat16)
    return jnp.dot(l, w1_ref[...], preferred_element_type=jnp.float32) \
        + b1_ref[...]


def _stats_body(coorst_ref, offc_ref, winc_ref, out_ref):
    i = pl.program_id(0)
    c = coorst_ref[...].astype(jnp.float32)
    t = c - offc_ref[...]
    t = t / winc_ref[...]
    t = t * 2.0
    t = t * 3.1415
    lb = t.astype(jnp.bfloat16).astype(jnp.float32)

    @pl.when(i == 0)
    def _init():
        out_ref[...] = jnp.zeros_like(out_ref)

    for j in range(3):
        out_ref[3:4, j:j + 1] += jnp.sum(
            lb[:, j, :], axis=1, keepdims=True)
        for k in range(j, 3):
            p = jnp.sum(lb[:, j, :] * lb[:, k, :], axis=1, keepdims=True)
            out_ref[j:j + 1, k:k + 1] += p


def _main_body(coors_ref, feat_ref, off_ref, win_ref, w1_ref, b1_ref,
               mean_ref, scale_ref, beta_ref, w2_ref, b2_ref, out_ref):
    h = _h(coors_ref, off_ref, win_ref, w1_ref, b1_ref)
    hn = (h - mean_ref[...]) * scale_ref[...] + beta_ref[...]
    u = jnp.maximum(hn, 0.0).astype(jnp.bfloat16)
    acc = jnp.dot(u, w2_ref[...], preferred_element_type=jnp.float32)
    out_ref[...] = acc + b2_ref[...] + feat_ref[...]


def kernel(features, coors, W1, b1, gamma, beta, W2, b2):
    n = features.shape[0]
    nf = jnp.float32(n)

    w1r = W1[::-1].astype(jnp.bfloat16)
    off = jnp.array([[20.5, 720.0, 720.0]], dtype=jnp.float32)
    win = jnp.array([[41.0, 1440.0, 1440.0]], dtype=jnp.float32)
    b1r = b1[None, :]

    common_specs = [
        pl.BlockSpec((1, 3), lambda i: (0, 0)),
        pl.BlockSpec((1, 3), lambda i: (0, 0)),
        pl.BlockSpec((3, _DIM), lambda i: (0, 0)),
        pl.BlockSpec((1, _DIM), lambda i: (0, 0)),
    ]

    nchunk = n // _STATS_TILE
    coorst = coors.reshape(nchunk, _STATS_TILE, 3).transpose(0, 2, 1)
    offc = off.reshape(1, 3, 1)
    winc = win.reshape(1, 3, 1)
    stats = pl.pallas_call(
        _stats_body,
        grid=(nchunk,),
        in_specs=[
            pl.BlockSpec((1, 3, _STATS_TILE), lambda i: (i, 0, 0)),
            pl.BlockSpec((1, 3, 1), lambda i: (0, 0, 0)),
            pl.BlockSpec((1, 3, 1), lambda i: (0, 0, 0)),
        ],
        out_specs=pl.BlockSpec((4, 3), lambda i: (0, 0)),
        out_shape=jax.ShapeDtypeStruct((4, 3), jnp.float32),
        compiler_params=pltpu.CompilerParams(
            dimension_semantics=("arbitrary",)),
    )(coorst, offc, winc)

    m = stats[3, :] / nf
    mom = stats[0:3, :] / nf
    sym = mom + mom.T - jnp.diag(jnp.diag(mom))
    cov = sym - jnp.outer(m, m)
    w1f = w1r.astype(jnp.float32)
    mean = (jnp.dot(m, w1f, precision=jax.lax.Precision.HIGHEST)
            + b1)[None, :]
    var = jnp.einsum("jc,jk,kc->c", w1f, cov, w1f,
                     precision=jax.lax.Precision.HIGHEST)[None, :]
    scale = gamma[None, :] / jnp.sqrt(var + 1e-5)

    out = pl.pallas_call(
        _main_body,
        grid=(n // _MAIN_TILE,),
        in_specs=[
            pl.BlockSpec((_MAIN_TILE, 3), lambda i: (i, 0)),
            pl.BlockSpec((_MAIN_TILE, _DIM), lambda i: (i, 0)),
        ]
        + common_specs
        + [
            pl.BlockSpec((1, _DIM), lambda i: (0, 0)),
            pl.BlockSpec((1, _DIM), lambda i: (0, 0)),
            pl.BlockSpec((1, _DIM), lambda i: (0, 0)),
            pl.BlockSpec((_DIM, _DIM), lambda i: (0, 0)),
            pl.BlockSpec((1, _DIM), lambda i: (0, 0)),
        ],
        out_specs=pl.BlockSpec((_MAIN_TILE, _DIM), lambda i: (i, 0)),
        out_shape=jax.ShapeDtypeStruct((n, _DIM), jnp.float32),
        compiler_params=pltpu.CompilerParams(
            dimension_semantics=("parallel",)),
    )(coors, features, off, win, w1r, b1r, mean, scale, beta[None, :],
      W2.astype(jnp.bfloat16), b2[None, :])
    return out


# speedup vs baseline: 5.7218x; 1.0264x over previous
import jax
import jax.numpy as jnp
from jax.experimental import pallas as pl
from jax.experimental.pallas import tpu as pltpu

_DIM = 256
_STATS_TILE = 8000
_MAIN_TILE = 8000


def _location(coors_ref, off_ref, win_ref):
    c = coors_ref[...].astype(jnp.float32)
    t = c - off_ref[...]
    t = t / win_ref[...]
    t = t * 2.0
    return t * 3.1415


def _h(coors_ref, off_ref, win_ref, w1_ref, b1_ref):
    l = _location(coors_ref, off_ref, win_ref).astype(jnp.bfloat16)
    return jnp.dot(l, w1_ref[...], preferred_element_type=jnp.float32) \
        + b1_ref[...]


def _stats_body(coorst_ref, offc_ref, winc_ref, out_ref):
    i = pl.program_id(0)
    c = coorst_ref[...].astype(jnp.float32)
    t = c - offc_ref[...]
    t = t / winc_ref[...]
    t = t * 2.0
    t = t * 3.1415
    lb = t.astype(jnp.bfloat16).astype(jnp.float32)

    @pl.when(i == 0)
    def _init():
        out_ref[...] = jnp.zeros_like(out_ref)

    for j in range(3):
        out_ref[3:4, j:j + 1] += jnp.sum(
            lb[:, j, :], axis=1, keepdims=True)
        for k in range(j, 3):
            p = jnp.sum(lb[:, j, :] * lb[:, k, :], axis=1, keepdims=True)
            out_ref[j:j + 1, k:k + 1] += p


def _main_body(coors_ref, feat_ref, off_ref, win_ref, w1_ref, b1_ref,
               mean_ref, scale_ref, beta_ref, w2_ref, b2_ref, out_ref):
    h = _h(coors_ref, off_ref, win_ref, w1_ref, b1_ref)
    hn = (h - mean_ref[...]) * scale_ref[...] + beta_ref[...]
    u = jnp.maximum(hn, 0.0).astype(jnp.bfloat16)
    acc = jnp.dot(u, w2_ref[...], preferred_element_type=jnp.float32)
    out_ref[...] = acc + b2_ref[...] + feat_ref[...]


def kernel(features, coors, W1, b1, gamma, beta, W2, b2):
    n = features.shape[0]
    nf = jnp.float32(n)

    w1r = W1[::-1].astype(jnp.bfloat16)
    off = jnp.array([[20.5, 720.0, 720.0]], dtype=jnp.float32)
    win = jnp.array([[41.0, 1440.0, 1440.0]], dtype=jnp.float32)
    b1r = b1[None, :]

    common_specs = [
        pl.BlockSpec((1, 3), lambda i: (0, 0)),
        pl.BlockSpec((1, 3), lambda i: (0, 0)),
        pl.BlockSpec((3, _DIM), lambda i: (0, 0)),
        pl.BlockSpec((1, _DIM), lambda i: (0, 0)),
    ]

    nchunk = n // _STATS_TILE
    coorst = coors.reshape(nchunk, _STATS_TILE, 3).transpose(0, 2, 1)
    offc = off.reshape(1, 3, 1)
    winc = win.reshape(1, 3, 1)
    stats = pl.pallas_call(
        _stats_body,
        grid=(nchunk,),
        in_specs=[
            pl.BlockSpec((1, 3, _STATS_TILE), lambda i: (i, 0, 0)),
            pl.BlockSpec((1, 3, 1), lambda i: (0, 0, 0)),
            pl.BlockSpec((1, 3, 1), lambda i: (0, 0, 0)),
        ],
        out_specs=pl.BlockSpec((4, 3), lambda i: (0, 0)),
        out_shape=jax.ShapeDtypeStruct((4, 3), jnp.float32),
        compiler_params=pltpu.CompilerParams(
            dimension_semantics=("arbitrary",)),
    )(coorst, offc, winc)

    m = stats[3, :] / nf
    mom = stats[0:3, :] / nf
    sym = mom + mom.T - jnp.diag(jnp.diag(mom))
    cov = sym - jnp.outer(m, m)
    w1f = w1r.astype(jnp.float32)
    mean = (jnp.dot(m, w1f, precision=jax.lax.Precision.HIGHEST)
            + b1)[None, :]
    var = jnp.einsum("jc,jk,kc->c", w1f, cov, w1f,
                     precision=jax.lax.Precision.HIGHEST)[None, :]
    scale = gamma[None, :] / jnp.sqrt(var + 1e-5)

    out = pl.pallas_call(
        _main_body,
        grid=(n // _MAIN_TILE,),
        in_specs=[
            pl.BlockSpec((_MAIN_TILE, 3), lambda i: (i, 0)),
            pl.BlockSpec((_MAIN_TILE, _DIM), lambda i: (i, 0)),
        ]
        + common_specs
        + [
            pl.BlockSpec((1, _DIM), lambda i: (0, 0)),
            pl.BlockSpec((1, _DIM), lambda i: (0, 0)),
            pl.BlockSpec((1, _DIM), lambda i: (0, 0)),
            pl.BlockSpec((_DIM, _DIM), lambda i: (0, 0)),
            pl.BlockSpec((1, _DIM), lambda i: (0, 0)),
        ],
        out_specs=pl.BlockSpec((_MAIN_TILE, _DIM), lambda i: (i, 0)),
        out_shape=jax.ShapeDtypeStruct((n, _DIM), jnp.float32),
        compiler_params=pltpu.CompilerParams(
            dimension_semantics=("parallel",)),
    )(coors, features, off, win, w1r, b1r, mean, scale, beta[None, :],
      W2.astype(jnp.bfloat16), b2[None, :])
    return out


# speedup vs baseline: 5.7512x; 1.0051x over previous
import jax
import jax.numpy as jnp
from jax.experimental import pallas as pl
from jax.experimental.pallas import tpu as pltpu

_DIM = 256
_STATS_TILE = 8000
_MAIN_TILE = 10000


def _location(coors_ref, off_ref, win_ref):
    c = coors_ref[...].astype(jnp.float32)
    t = c - off_ref[...]
    t = t / win_ref[...]
    t = t * 2.0
    return t * 3.1415


def _h(coors_ref, off_ref, win_ref, w1_ref, b1_ref):
    l = _location(coors_ref, off_ref, win_ref).astype(jnp.bfloat16)
    return jnp.dot(l, w1_ref[...], preferred_element_type=jnp.float32) \
        + b1_ref[...]


def _stats_body(coorst_ref, offc_ref, winc_ref, out_ref):
    i = pl.program_id(0)
    c = coorst_ref[...].astype(jnp.float32)
    t = c - offc_ref[...]
    t = t / winc_ref[...]
    t = t * 2.0
    t = t * 3.1415
    lb = t.astype(jnp.bfloat16).astype(jnp.float32)

    @pl.when(i == 0)
    def _init():
        out_ref[...] = jnp.zeros_like(out_ref)

    for j in range(3):
        out_ref[3:4, j:j + 1] += jnp.sum(
            lb[:, j, :], axis=1, keepdims=True)
        for k in range(j, 3):
            p = jnp.sum(lb[:, j, :] * lb[:, k, :], axis=1, keepdims=True)
            out_ref[j:j + 1, k:k + 1] += p


def _main_body(coors_ref, feat_ref, off_ref, win_ref, w1_ref, b1_ref,
               mean_ref, scale_ref, beta_ref, w2_ref, b2_ref, out_ref):
    h = _h(coors_ref, off_ref, win_ref, w1_ref, b1_ref)
    hn = (h - mean_ref[...]) * scale_ref[...] + beta_ref[...]
    u = jnp.maximum(hn, 0.0).astype(jnp.bfloat16)
    acc = jnp.dot(u, w2_ref[...], preferred_element_type=jnp.float32)
    out_ref[...] = acc + b2_ref[...] + feat_ref[...]


def kernel(features, coors, W1, b1, gamma, beta, W2, b2):
    n = features.shape[0]
    nf = jnp.float32(n)

    w1r = W1[::-1].astype(jnp.bfloat16)
    off = jnp.array([[20.5, 720.0, 720.0]], dtype=jnp.float32)
    win = jnp.array([[41.0, 1440.0, 1440.0]], dtype=jnp.float32)
    b1r = b1[None, :]

    common_specs = [
        pl.BlockSpec((1, 3), lambda i: (0, 0)),
        pl.BlockSpec((1, 3), lambda i: (0, 0)),
        pl.BlockSpec((3, _DIM), lambda i: (0, 0)),
        pl.BlockSpec((1, _DIM), lambda i: (0, 0)),
    ]

    nchunk = n // _STATS_TILE
    coorst = coors.reshape(nchunk, _STATS_TILE, 3).transpose(0, 2, 1)
    offc = off.reshape(1, 3, 1)
    winc = win.reshape(1, 3, 1)
    stats = pl.pallas_call(
        _stats_body,
        grid=(nchunk,),
        in_specs=[
            pl.BlockSpec((1, 3, _STATS_TILE), lambda i: (i, 0, 0)),
            pl.BlockSpec((1, 3, 1), lambda i: (0, 0, 0)),
            pl.BlockSpec((1, 3, 1), lambda i: (0, 0, 0)),
        ],
        out_specs=pl.BlockSpec((4, 3), lambda i: (0, 0)),
        out_shape=jax.ShapeDtypeStruct((4, 3), jnp.float32),
        compiler_params=pltpu.CompilerParams(
            dimension_semantics=("arbitrary",)),
    )(coorst, offc, winc)

    m = stats[3, :] / nf
    mom = stats[0:3, :] / nf
    sym = mom + mom.T - jnp.diag(jnp.diag(mom))
    cov = sym - jnp.outer(m, m)
    w1f = w1r.astype(jnp.float32)
    mean = (jnp.dot(m, w1f, precision=jax.lax.Precision.HIGHEST)
            + b1)[None, :]
    var = jnp.einsum("jc,jk,kc->c", w1f, cov, w1f,
                     precision=jax.lax.Precision.HIGHEST)[None, :]
    scale = gamma[None, :] / jnp.sqrt(var + 1e-5)

    out = pl.pallas_call(
        _main_body,
        grid=(n // _MAIN_TILE,),
        in_specs=[
            pl.BlockSpec((_MAIN_TILE, 3), lambda i: (i, 0)),
            pl.BlockSpec((_MAIN_TILE, _DIM), lambda i: (i, 0)),
        ]
        + common_specs
        + [
            pl.BlockSpec((1, _DIM), lambda i: (0, 0)),
            pl.BlockSpec((1, _DIM), lambda i: (0, 0)),
            pl.BlockSpec((1, _DIM), lambda i: (0, 0)),
            pl.BlockSpec((_DIM, _DIM), lambda i: (0, 0)),
            pl.BlockSpec((1, _DIM), lambda i: (0, 0)),
        ],
        out_specs=pl.BlockSpec((_MAIN_TILE, _DIM), lambda i: (i, 0)),
        out_shape=jax.ShapeDtypeStruct((n, _DIM), jnp.float32),
        compiler_params=pltpu.CompilerParams(
            dimension_semantics=("parallel",)),
    )(coors, features, off, win, w1r, b1r, mean, scale, beta[None, :],
      W2.astype(jnp.bfloat16), b2[None, :])
    return out


# speedup vs baseline: 5.8746x; 1.0214x over previous
import jax
import jax.numpy as jnp
from jax.experimental import pallas as pl
from jax.experimental.pallas import tpu as pltpu

_DIM = 256
_STATS_TILE = 8000
_MAIN_TILE = 10000


def _location(coors_ref, off_ref, win_ref):
    c = coors_ref[...].astype(jnp.float32)
    t = c - off_ref[...]
    t = t / win_ref[...]
    t = t * 2.0
    return t * 3.1415


def _h(coors_ref, off_ref, win_ref, w1_ref, b1_ref):
    l = _location(coors_ref, off_ref, win_ref).astype(jnp.bfloat16)
    return jnp.dot(l, w1_ref[...], preferred_element_type=jnp.float32) \
        + b1_ref[...]


def _stats_body(coorst_ref, offc_ref, winc_ref, w1f_ref, b1_ref,
                gamma_ref, mom_ref, mean_ref, scale_ref, *,
                nrows, nprog):
    i = pl.program_id(0)
    c = coorst_ref[...].astype(jnp.float32)
    t = c - offc_ref[...]
    t = t / winc_ref[...]
    t = t * 2.0
    t = t * 3.1415
    lb = t.astype(jnp.bfloat16).astype(jnp.float32)

    @pl.when(i == 0)
    def _init():
        mom_ref[...] = jnp.zeros_like(mom_ref)

    for j in range(3):
        mom_ref[3:4, j:j + 1] += jnp.sum(
            lb[:, j, :], axis=1, keepdims=True)
        for k in range(j, 3):
            p = jnp.sum(lb[:, j, :] * lb[:, k, :], axis=1, keepdims=True)
            mom_ref[j:j + 1, k:k + 1] += p

    @pl.when(i == nprog - 1)
    def _finish():
        inv_n = 1.0 / nrows
        m = [mom_ref[3, j] * inv_n for j in range(3)]
        mean = b1_ref[...]
        var = jnp.zeros((1, _DIM), jnp.float32)
        for j in range(3):
            mean = mean + m[j] * w1f_ref[j:j + 1, :]
            for k in range(j, 3):
                cjk = mom_ref[j, k] * inv_n - m[j] * m[k]
                w = 1.0 if j == k else 2.0
                var = var + (w * cjk) * (w1f_ref[j:j + 1, :]
                                         * w1f_ref[k:k + 1, :])
        mean_ref[...] = mean
        scale_ref[...] = gamma_ref[...] / jnp.sqrt(var + 1e-5)


def _main_body(coors_ref, feat_ref, off_ref, win_ref, w1_ref, b1_ref,
               mean_ref, scale_ref, beta_ref, w2_ref, b2_ref, out_ref):
    h = _h(coors_ref, off_ref, win_ref, w1_ref, b1_ref)
    hn = (h - mean_ref[...]) * scale_ref[...] + beta_ref[...]
    u = jnp.maximum(hn, 0.0).astype(jnp.bfloat16)
    acc = jnp.dot(u, w2_ref[...], preferred_element_type=jnp.float32)
    out_ref[...] = acc + b2_ref[...] + feat_ref[...]


def kernel(features, coors, W1, b1, gamma, beta, W2, b2):
    n = features.shape[0]
    nf = jnp.float32(n)

    w1r = W1[::-1].astype(jnp.bfloat16)
    off = jnp.array([[20.5, 720.0, 720.0]], dtype=jnp.float32)
    win = jnp.array([[41.0, 1440.0, 1440.0]], dtype=jnp.float32)
    b1r = b1[None, :]

    common_specs = [
        pl.BlockSpec((1, 3), lambda i: (0, 0)),
        pl.BlockSpec((1, 3), lambda i: (0, 0)),
        pl.BlockSpec((3, _DIM), lambda i: (0, 0)),
        pl.BlockSpec((1, _DIM), lambda i: (0, 0)),
    ]

    import functools as _ft
    nchunk = n // _STATS_TILE
    coorst = coors.reshape(nchunk, _STATS_TILE, 3).transpose(0, 2, 1)
    offc = off.reshape(1, 3, 1)
    winc = win.reshape(1, 3, 1)
    w1f = w1r.astype(jnp.float32)
    vec_spec = pl.BlockSpec((1, _DIM), lambda i: (0, 0))
    _, mean, scale = pl.pallas_call(
        _ft.partial(_stats_body, nrows=float(n), nprog=nchunk),
        grid=(nchunk,),
        in_specs=[
            pl.BlockSpec((1, 3, _STATS_TILE), lambda i: (i, 0, 0)),
            pl.BlockSpec((1, 3, 1), lambda i: (0, 0, 0)),
            pl.BlockSpec((1, 3, 1), lambda i: (0, 0, 0)),
            pl.BlockSpec((3, _DIM), lambda i: (0, 0)),
            vec_spec,
            vec_spec,
        ],
        out_specs=[
            pl.BlockSpec((4, 3), lambda i: (0, 0)),
            vec_spec,
            vec_spec,
        ],
        out_shape=[
            jax.ShapeDtypeStruct((4, 3), jnp.float32),
            jax.ShapeDtypeStruct((1, _DIM), jnp.float32),
            jax.ShapeDtypeStruct((1, _DIM), jnp.float32),
        ],
        compiler_params=pltpu.CompilerParams(
            dimension_semantics=("arbitrary",)),
    )(coorst, offc, winc, w1f, b1r, gamma[None, :])

    out = pl.pallas_call(
        _main_body,
        grid=(n // _MAIN_TILE,),
        in_specs=[
            pl.BlockSpec((_MAIN_TILE, 3), lambda i: (i, 0)),
            pl.BlockSpec((_MAIN_TILE, _DIM), lambda i: (i, 0)),
        ]
        + common_specs
        + [
            pl.BlockSpec((1, _DIM), lambda i: (0, 0)),
            pl.BlockSpec((1, _DIM), lambda i: (0, 0)),
            pl.BlockSpec((1, _DIM), lambda i: (0, 0)),
            pl.BlockSpec((_DIM, _DIM), lambda i: (0, 0)),
            pl.BlockSpec((1, _DIM), lambda i: (0, 0)),
        ],
        out_specs=pl.BlockSpec((_MAIN_TILE, _DIM), lambda i: (i, 0)),
        out_shape=jax.ShapeDtypeStruct((n, _DIM), jnp.float32),
        compiler_params=pltpu.CompilerParams(
            dimension_semantics=("parallel",)),
    )(coors, features, off, win, w1r, b1r, mean, scale, beta[None, :],
      W2.astype(jnp.bfloat16), b2[None, :])
    return out


# speedup vs baseline: 6.0736x; 1.0339x over previous
import jax
import jax.numpy as jnp
from jax.experimental import pallas as pl
from jax.experimental.pallas import tpu as pltpu

_DIM = 256
_STATS_TILE = 25000
_MAIN_TILE = 10000


def _location(coors_ref, off_ref, win_ref):
    c = coors_ref[...].astype(jnp.float32)
    t = c - off_ref[...]
    t = t / win_ref[...]
    t = t * 2.0
    return t * 3.1415


def _h(coors_ref, off_ref, win_ref, w1_ref, b1_ref):
    l = _location(coors_ref, off_ref, win_ref).astype(jnp.bfloat16)
    return jnp.dot(l, w1_ref[...], preferred_element_type=jnp.float32) \
        + b1_ref[...]


def _stats_body(coorst_ref, offc_ref, winc_ref, w1f_ref, b1_ref,
                gamma_ref, mom_ref, mean_ref, scale_ref, *,
                nrows, nprog):
    i = pl.program_id(0)
    c = coorst_ref[...].astype(jnp.float32)
    t = c - offc_ref[...]
    t = t / winc_ref[...]
    t = t * 2.0
    t = t * 3.1415
    lb = t.astype(jnp.bfloat16).astype(jnp.float32)

    @pl.when(i == 0)
    def _init():
        mom_ref[...] = jnp.zeros_like(mom_ref)

    for j in range(3):
        mom_ref[3:4, j:j + 1] += jnp.sum(
            lb[:, j, :], axis=1, keepdims=True)
        for k in range(j, 3):
            p = jnp.sum(lb[:, j, :] * lb[:, k, :], axis=1, keepdims=True)
            mom_ref[j:j + 1, k:k + 1] += p

    @pl.when(i == nprog - 1)
    def _finish():
        inv_n = 1.0 / nrows
        m = [mom_ref[3, j] * inv_n for j in range(3)]
        mean = b1_ref[...]
        var = jnp.zeros((1, _DIM), jnp.float32)
        for j in range(3):
            mean = mean + m[j] * w1f_ref[j:j + 1, :]
            for k in range(j, 3):
                cjk = mom_ref[j, k] * inv_n - m[j] * m[k]
                w = 1.0 if j == k else 2.0
                var = var + (w * cjk) * (w1f_ref[j:j + 1, :]
                                         * w1f_ref[k:k + 1, :])
        mean_ref[...] = mean
        scale_ref[...] = gamma_ref[...] / jnp.sqrt(var + 1e-5)


def _main_body(coors_ref, feat_ref, off_ref, win_ref, w1_ref, b1_ref,
               mean_ref, scale_ref, beta_ref, w2_ref, b2_ref, out_ref):
    h = _h(coors_ref, off_ref, win_ref, w1_ref, b1_ref)
    hn = (h - mean_ref[...]) * scale_ref[...] + beta_ref[...]
    u = jnp.maximum(hn, 0.0).astype(jnp.bfloat16)
    acc = jnp.dot(u, w2_ref[...], preferred_element_type=jnp.float32)
    out_ref[...] = acc + b2_ref[...] + feat_ref[...]


def kernel(features, coors, W1, b1, gamma, beta, W2, b2):
    n = features.shape[0]
    nf = jnp.float32(n)

    w1r = W1[::-1].astype(jnp.bfloat16)
    off = jnp.array([[20.5, 720.0, 720.0]], dtype=jnp.float32)
    win = jnp.array([[41.0, 1440.0, 1440.0]], dtype=jnp.float32)
    b1r = b1[None, :]

    common_specs = [
        pl.BlockSpec((1, 3), lambda i: (0, 0)),
        pl.BlockSpec((1, 3), lambda i: (0, 0)),
        pl.BlockSpec((3, _DIM), lambda i: (0, 0)),
        pl.BlockSpec((1, _DIM), lambda i: (0, 0)),
    ]

    import functools as _ft
    nchunk = n // _STATS_TILE
    coorst = coors.reshape(nchunk, _STATS_TILE, 3).transpose(0, 2, 1)
    offc = off.reshape(1, 3, 1)
    winc = win.reshape(1, 3, 1)
    w1f = w1r.astype(jnp.float32)
    vec_spec = pl.BlockSpec((1, _DIM), lambda i: (0, 0))
    _, mean, scale = pl.pallas_call(
        _ft.partial(_stats_body, nrows=float(n), nprog=nchunk),
        grid=(nchunk,),
        in_specs=[
            pl.BlockSpec((1, 3, _STATS_TILE), lambda i: (i, 0, 0)),
            pl.BlockSpec((1, 3, 1), lambda i: (0, 0, 0)),
            pl.BlockSpec((1, 3, 1), lambda i: (0, 0, 0)),
            pl.BlockSpec((3, _DIM), lambda i: (0, 0)),
            vec_spec,
            vec_spec,
        ],
        out_specs=[
            pl.BlockSpec((4, 3), lambda i: (0, 0)),
            vec_spec,
            vec_spec,
        ],
        out_shape=[
            jax.ShapeDtypeStruct((4, 3), jnp.float32),
            jax.ShapeDtypeStruct((1, _DIM), jnp.float32),
            jax.ShapeDtypeStruct((1, _DIM), jnp.float32),
        ],
        compiler_params=pltpu.CompilerParams(
            dimension_semantics=("arbitrary",)),
    )(coorst, offc, winc, w1f, b1r, gamma[None, :])

    out = pl.pallas_call(
        _main_body,
        grid=(n // _MAIN_TILE,),
        in_specs=[
            pl.BlockSpec((_MAIN_TILE, 3), lambda i: (i, 0)),
            pl.BlockSpec((_MAIN_TILE, _DIM), lambda i: (i, 0)),
        ]
        + common_specs
        + [
            pl.BlockSpec((1, _DIM), lambda i: (0, 0)),
            pl.BlockSpec((1, _DIM), lambda i: (0, 0)),
            pl.BlockSpec((1, _DIM), lambda i: (0, 0)),
            pl.BlockSpec((_DIM, _DIM), lambda i: (0, 0)),
            pl.BlockSpec((1, _DIM), lambda i: (0, 0)),
        ],
        out_specs=pl.BlockSpec((_MAIN_TILE, _DIM), lambda i: (i, 0)),
        out_shape=jax.ShapeDtypeStruct((n, _DIM), jnp.float32),
        compiler_params=pltpu.CompilerParams(
            dimension_semantics=("parallel",)),
    )(coors, features, off, win, w1r, b1r, mean, scale, beta[None, :],
      W2.astype(jnp.bfloat16), b2[None, :])
    return out


# speedup vs baseline: 6.0996x; 1.0043x over previous
import jax
import jax.numpy as jnp
from jax.experimental import pallas as pl
from jax.experimental.pallas import tpu as pltpu

_DIM = 256
_STATS_TILE = 40000
_MAIN_TILE = 10000


def _location(coors_ref, off_ref, win_ref):
    c = coors_ref[...].astype(jnp.float32)
    t = c - off_ref[...]
    t = t / win_ref[...]
    t = t * 2.0
    return t * 3.1415


def _h(coors_ref, off_ref, win_ref, w1_ref, b1_ref):
    l = _location(coors_ref, off_ref, win_ref).astype(jnp.bfloat16)
    return jnp.dot(l, w1_ref[...], preferred_element_type=jnp.float32) \
        + b1_ref[...]


def _stats_body(coorst_ref, offc_ref, winc_ref, w1f_ref, b1_ref,
                gamma_ref, mom_ref, mean_ref, scale_ref, *,
                nrows, nprog):
    i = pl.program_id(0)
    c = coorst_ref[...].astype(jnp.float32)
    t = c - offc_ref[...]
    t = t / winc_ref[...]
    t = t * 2.0
    t = t * 3.1415
    lb = t.astype(jnp.bfloat16).astype(jnp.float32)

    @pl.when(i == 0)
    def _init():
        mom_ref[...] = jnp.zeros_like(mom_ref)

    for j in range(3):
        mom_ref[3:4, j:j + 1] += jnp.sum(
            lb[:, j, :], axis=1, keepdims=True)
        for k in range(j, 3):
            p = jnp.sum(lb[:, j, :] * lb[:, k, :], axis=1, keepdims=True)
            mom_ref[j:j + 1, k:k + 1] += p

    @pl.when(i == nprog - 1)
    def _finish():
        inv_n = 1.0 / nrows
        m = [mom_ref[3, j] * inv_n for j in range(3)]
        mean = b1_ref[...]
        var = jnp.zeros((1, _DIM), jnp.float32)
        for j in range(3):
            mean = mean + m[j] * w1f_ref[j:j + 1, :]
            for k in range(j, 3):
                cjk = mom_ref[j, k] * inv_n - m[j] * m[k]
                w = 1.0 if j == k else 2.0
                var = var + (w * cjk) * (w1f_ref[j:j + 1, :]
                                         * w1f_ref[k:k + 1, :])
        mean_ref[...] = mean
        scale_ref[...] = gamma_ref[...] / jnp.sqrt(var + 1e-5)


def _main_body(coors_ref, feat_ref, off_ref, win_ref, w1_ref, b1_ref,
               mean_ref, scale_ref, beta_ref, w2_ref, b2_ref, out_ref):
    h = _h(coors_ref, off_ref, win_ref, w1_ref, b1_ref)
    hn = (h - mean_ref[...]) * scale_ref[...] + beta_ref[...]
    u = jnp.maximum(hn, 0.0).astype(jnp.bfloat16)
    acc = jnp.dot(u, w2_ref[...], preferred_element_type=jnp.float32)
    out_ref[...] = acc + b2_ref[...] + feat_ref[...]


def kernel(features, coors, W1, b1, gamma, beta, W2, b2):
    n = features.shape[0]
    nf = jnp.float32(n)

    w1r = W1[::-1].astype(jnp.bfloat16)
    off = jnp.array([[20.5, 720.0, 720.0]], dtype=jnp.float32)
    win = jnp.array([[41.0, 1440.0, 1440.0]], dtype=jnp.float32)
    b1r = b1[None, :]

    common_specs = [
        pl.BlockSpec((1, 3), lambda i: (0, 0)),
        pl.BlockSpec((1, 3), lambda i: (0, 0)),
        pl.BlockSpec((3, _DIM), lambda i: (0, 0)),
        pl.BlockSpec((1, _DIM), lambda i: (0, 0)),
    ]

    import functools as _ft
    nchunk = n // _STATS_TILE
    coorst = coors.reshape(nchunk, _STATS_TILE, 3).transpose(0, 2, 1)
    offc = off.reshape(1, 3, 1)
    winc = win.reshape(1, 3, 1)
    w1f = w1r.astype(jnp.float32)
    vec_spec = pl.BlockSpec((1, _DIM), lambda i: (0, 0))
    _, mean, scale = pl.pallas_call(
        _ft.partial(_stats_body, nrows=float(n), nprog=nchunk),
        grid=(nchunk,),
        in_specs=[
            pl.BlockSpec((1, 3, _STATS_TILE), lambda i: (i, 0, 0)),
            pl.BlockSpec((1, 3, 1), lambda i: (0, 0, 0)),
            pl.BlockSpec((1, 3, 1), lambda i: (0, 0, 0)),
            pl.BlockSpec((3, _DIM), lambda i: (0, 0)),
            vec_spec,
            vec_spec,
        ],
        out_specs=[
            pl.BlockSpec((4, 3), lambda i: (0, 0)),
            vec_spec,
            vec_spec,
        ],
        out_shape=[
            jax.ShapeDtypeStruct((4, 3), jnp.float32),
            jax.ShapeDtypeStruct((1, _DIM), jnp.float32),
            jax.ShapeDtypeStruct((1, _DIM), jnp.float32),
        ],
        compiler_params=pltpu.CompilerParams(
            dimension_semantics=("arbitrary",)),
    )(coorst, offc, winc, w1f, b1r, gamma[None, :])

    out = pl.pallas_call(
        _main_body,
        grid=(n // _MAIN_TILE,),
        in_specs=[
            pl.BlockSpec((_MAIN_TILE, 3), lambda i: (i, 0)),
            pl.BlockSpec((_MAIN_TILE, _DIM), lambda i: (i, 0)),
        ]
        + common_specs
        + [
            pl.BlockSpec((1, _DIM), lambda i: (0, 0)),
            pl.BlockSpec((1, _DIM), lambda i: (0, 0)),
            pl.BlockSpec((1, _DIM), lambda i: (0, 0)),
            pl.BlockSpec((_DIM, _DIM), lambda i: (0, 0)),
            pl.BlockSpec((1, _DIM), lambda i: (0, 0)),
        ],
        out_specs=pl.BlockSpec((_MAIN_TILE, _DIM), lambda i: (i, 0)),
        out_shape=jax.ShapeDtypeStruct((n, _DIM), jnp.float32),
        compiler_params=pltpu.CompilerParams(
            dimension_semantics=("parallel",)),
    )(coors, features, off, win, w1r, b1r, mean, scale, beta[None, :],
      W2.astype(jnp.bfloat16), b2[None, :])
    return out
